# baseline (speedup 1.0000x reference)
import functools

import jax
import jax.numpy as jnp
from jax import lax
from jax.experimental import pallas as pl
from jax.experimental.pallas import tpu as pltpu
from jax.experimental.pallas import tpu_sc as plsc

B = 16384
V = 100000
D0 = 32
D1 = 64
DC = D0 + D1
NC = 2
NS = 16
NW = NC * NS
CHUNK = 128
NW_ROWS = B // NW
CPW = NW_ROWS // CHUNK
L = 16

_mesh = plsc.VectorSubcoreMesh(core_axis_name="c", subcore_axis_name="s")


@functools.partial(
    pl.kernel,
    mesh=_mesh,
    out_type=[
        jax.ShapeDtypeStruct((B, DC), jnp.float32),
        jax.ShapeDtypeStruct((B, DC), jnp.float32),
    ],
    scratch_types=[
        pltpu.VMEM((NW_ROWS,), jnp.int32),
        pltpu.VMEM((NW_ROWS,), jnp.int32),
        pltpu.VMEM((NW_ROWS, D0), jnp.float32),
        pltpu.VMEM((NW_ROWS, D1), jnp.float32),
        pltpu.VMEM((NW_ROWS, D0), jnp.float32),
        pltpu.VMEM((NW_ROWS, D1), jnp.float32),
        pltpu.SemaphoreType.DMA,
        pltpu.SemaphoreType.DMA,
        pltpu.SemaphoreType.DMA,
    ],
    compiler_params=pltpu.CompilerParams(use_tc_tiling_on_sc=False),
)
def _mixed_emb(t0, t1, iid, uid, item_out, user_out,
               iidx_v, uidx_v, vi0, vi1, vu0, vu1, s_i, s_u, s_w):
    wid = lax.axis_index("s") * NC + lax.axis_index("c")
    base = wid * NW_ROWS
    pltpu.sync_copy(iid.at[pl.ds(base, NW_ROWS)], iidx_v)
    pltpu.sync_copy(uid.at[pl.ds(base, NW_ROWS)], uidx_v)
    for j in range(NW_ROWS // L):
        sl = pl.ds(j * L, L)
        uidx_v[sl] = uidx_v[sl] + V
    gathers = []
    for c in range(CPW):
        isl = pl.ds(c * CHUNK, CHUNK)
        rows = pl.ds(c * CHUNK, CHUNK)
        gathers.append((
            pltpu.async_copy(t0.at[iidx_v.at[isl]], vi0.at[rows], s_i),
            pltpu.async_copy(t1.at[iidx_v.at[isl]], vi1.at[rows], s_i),
            pltpu.async_copy(t0.at[uidx_v.at[isl]], vu0.at[rows], s_u),
            pltpu.async_copy(t1.at[uidx_v.at[isl]], vu1.at[rows], s_u),
        ))
    orows = pl.ds(base, NW_ROWS)
    for g in gathers:
        g[0].wait()
    w0 = pltpu.async_copy(vi0, item_out.at[orows, pl.ds(0, D0)], s_w)
    for g in gathers:
        g[1].wait()
    w1 = pltpu.async_copy(vi1, item_out.at[orows, pl.ds(D0, D1)], s_w)
    for g in gathers:
        g[2].wait()
    w2 = pltpu.async_copy(vu0, user_out.at[orows, pl.ds(0, D0)], s_w)
    for g in gathers:
        g[3].wait()
    w3 = pltpu.async_copy(vu1, user_out.at[orows, pl.ds(D0, D1)], s_w)
    w0.wait()
    w1.wait()
    w2.wait()
    w3.wait()


def kernel(item_table0, user_table0, item_table1, user_table1, item_ids, user_ids):
    t0 = jnp.concatenate([item_table0, user_table0], axis=0)
    t1 = jnp.concatenate([item_table1, user_table1], axis=0)
    return _mixed_emb(t0, t1, item_ids, user_ids)


# speedup vs baseline: 1.2678x; 1.2678x over previous
import functools

import jax
import jax.numpy as jnp
from jax import lax
from jax.experimental import pallas as pl
from jax.experimental.pallas import tpu as pltpu
from jax.experimental.pallas import tpu_sc as plsc

B = 16384
D0 = 32
D1 = 64
DC = D0 + D1
NC = 2
NS = 16
NW = NC * NS
CHUNK = 128
CPW = B // (NW * CHUNK)
NW_ROWS = CPW * CHUNK

_mesh = plsc.VectorSubcoreMesh(core_axis_name="c", subcore_axis_name="s")


def _make_gather(d):
    @functools.partial(
        pl.kernel,
        mesh=_mesh,
        out_type=jax.ShapeDtypeStruct((B, d), jnp.float32),
        scratch_types=[
            pltpu.VMEM((NW_ROWS,), jnp.int32),
            pltpu.VMEM((NW_ROWS, d), jnp.float32),
            pltpu.SemaphoreType.DMA,
            pltpu.SemaphoreType.DMA,
        ],
        compiler_params=pltpu.CompilerParams(use_tc_tiling_on_sc=False),
    )
    def _gather(t, ids, out, idx_v, v, s_g, s_w):
        wid = lax.axis_index("s") * NC + lax.axis_index("c")
        base = wid * NW_ROWS
        pltpu.sync_copy(ids.at[pl.ds(base, NW_ROWS)], idx_v)
        gathers = []
        for c in range(CPW):
            isl = pl.ds(c * CHUNK, CHUNK)
            rows = pl.ds(c * CHUNK, CHUNK)
            gathers.append(
                pltpu.async_copy(t.at[idx_v.at[isl]], v.at[rows], s_g))
        for g in gathers:
            g.wait()
        pltpu.async_copy(v, out.at[pl.ds(base, NW_ROWS)], s_w).wait()
    return _gather


_gather32 = _make_gather(D0)
_gather64 = _make_gather(D1)


def kernel(item_table0, user_table0, item_table1, user_table1, item_ids, user_ids):
    i0 = _gather32(item_table0, item_ids)
    i1 = _gather64(item_table1, item_ids)
    u0 = _gather32(user_table0, user_ids)
    u1 = _gather64(user_table1, user_ids)
    return (jnp.concatenate([i0, i1], axis=1),
            jnp.concatenate([u0, u1], axis=1))
